# baseline (speedup 1.0000x reference)
import functools

import jax
import jax.numpy as jnp
from jax import lax
from jax.experimental import pallas as pl
from jax.experimental.pallas import tpu as pltpu
from jax.experimental.pallas import tpu_sc as plsc

_GAMMA = 12.0
_HIDDEN = 64
_BATCH = 16384
_NW = 32
_BPW = _BATCH // _NW
_CHUNK = 128
_NCHUNK = _BPW // _CHUNK
_GROUPS = _BPW // 16


def _sqrt16(x):
    xs = jnp.maximum(x, jnp.float32(1.1754944e-38))
    i = lax.bitcast_convert_type(xs, jnp.int32)
    i = jnp.int32(0x5F3759DF) - lax.shift_right_arithmetic(i, jnp.int32(1))
    y = lax.bitcast_convert_type(i, jnp.float32)
    for _ in range(3):
        y = y * (jnp.float32(1.5) - jnp.float32(0.5) * xs * y * y)
    return xs * y


def _make_sc_kernel():
    mesh = plsc.VectorSubcoreMesh(core_axis_name="c", subcore_axis_name="s")

    @functools.partial(
        pl.kernel,
        mesh=mesh,
        compiler_params=pltpu.CompilerParams(
            needs_layout_passes=False, use_tc_tiling_on_sc=False),
        out_type=jax.ShapeDtypeStruct((_NW, _BPW), jnp.float32),
        scratch_types=[
            pltpu.VMEM((_NCHUNK, _CHUNK), jnp.int32),
            pltpu.VMEM((_NCHUNK, _CHUNK), jnp.int32),
            pltpu.VMEM((_BPW, _HIDDEN), jnp.float32),
            pltpu.VMEM((_BPW, _HIDDEN), jnp.float32),
            pltpu.VMEM((_BPW,), jnp.float32),
            pltpu.SemaphoreType.DMA,
        ],
    )
    def sc_kernel(table_hbm, hidx_hbm, tidx_hbm, out_hbm,
                  hidx_v, tidx_v, hrows_v, trows_v, scores_v, sem):
        wid = lax.axis_index("s") * 2 + lax.axis_index("c")

        pltpu.sync_copy(hidx_hbm.at[wid], hidx_v)
        pltpu.sync_copy(tidx_hbm.at[wid], tidx_v)

        copies = []
        for j in range(_NCHUNK):
            copies.append(pltpu.async_copy(
                table_hbm.at[hidx_v.at[j]],
                hrows_v.at[pl.ds(j * _CHUNK, _CHUNK)], sem))
            copies.append(pltpu.async_copy(
                table_hbm.at[tidx_v.at[j]],
                trows_v.at[pl.ds(j * _CHUNK, _CHUNK)], sem))
        for c in copies:
            c.wait()

        iota16 = lax.iota(jnp.int32, 16)

        def group(g, carry):
            sums = jnp.zeros((16,), jnp.float32)
            for j in range(16):
                r = g * 16 + j
                hrow = hrows_v.at[r]
                trow = trows_v.at[r]
                accs = []
                for k in range(_HIDDEN // 16):
                    h = hrow[pl.ds(k * 16, 16)]
                    t = trow[pl.ds(k * 16, 16)]
                    df = h - t
                    accs.append(df * df)
                s = (accs[0] + accs[1]) + (accs[2] + accs[3])
                total = jnp.sum(s)
                sums = jnp.where(iota16 == j, total, sums)
            scores_v[pl.ds(g * 16, 16)] = jnp.float32(_GAMMA) - _sqrt16(sums)
            return carry

        lax.fori_loop(0, _GROUPS, group, 0)

        pltpu.sync_copy(scores_v, out_hbm.at[wid])

    return sc_kernel


_sc_kernel = _make_sc_kernel()


@jax.jit
def kernel(sample, entity_embedding):
    hidx = sample[:, 0].reshape(_NW, _NCHUNK, _CHUNK)
    tidx = sample[:, 1].reshape(_NW, _NCHUNK, _CHUNK)
    out = _sc_kernel(entity_embedding, hidx, tidx)
    return out.reshape(_BATCH, 1)
